# Initial kernel scaffold; baseline (speedup 1.0000x reference)
import jax
import jax.numpy as jnp
from jax.experimental import pallas as pl


def kernel(x, edge_index, dt, norm, decay_lam):
    raise NotImplementedError("write your pallas kernel here")


# speedup vs baseline: 3.1541x; 3.1541x over previous
import functools

import jax
import jax.numpy as jnp
from jax import lax
from jax.experimental import pallas as pl
from jax.experimental.pallas import tpu as pltpu
from jax.experimental.pallas import tpu_sc as plsc

D_FEAT = 128
NC = 2
NS = 16
NW = NC * NS
CHUNK = 128


def _sc_body(x_hbm, src_hbm, dst_hbm, dt_hbm, norm_hbm, lam_hbm, out_hbm,
             src_v, dst_v, dt_v, norm_v, rows_v, lam_v, acc, sem,
             *, chunks_per_tile, rows_per_tile):
    cid = lax.axis_index("c")
    sid = lax.axis_index("s")
    wid = sid * NC + cid

    def zrow(e, carry):
        for cc in range(D_FEAT // 16):
            rows_v[e, pl.ds(cc * 16, 16)] = jnp.zeros((16,), jnp.float32)
        return carry
    lax.fori_loop(0, CHUNK, zrow, 0)
    for q in range(rows_per_tile // CHUNK):
        pltpu.sync_copy(rows_v,
                        acc.at[pl.ds(sid * rows_per_tile + q * CHUNK, CHUNK)])
    plsc.subcore_barrier()

    pltpu.sync_copy(lam_hbm, lam_v)
    neg_lam = -(jnp.maximum(lam_v[...], 0.0) + 1e-4)

    edges_per_tile = chunks_per_tile * CHUNK

    def chunk_body(ci, carry):
        base = wid * edges_per_tile + ci * CHUNK
        pltpu.sync_copy(src_hbm.at[pl.ds(base, CHUNK)], src_v)
        pltpu.sync_copy(dst_hbm.at[pl.ds(base, CHUNK)], dst_v)
        pltpu.sync_copy(dt_hbm.at[pl.ds(base, CHUNK)], dt_v)
        pltpu.sync_copy(norm_hbm.at[pl.ds(base, CHUNK)], norm_v)
        pltpu.async_copy(x_hbm.at[src_v], rows_v, sem).wait()
        def scale(g, c2):
            sl = pl.ds(g * 16, 16)
            w16 = norm_v[sl] * jnp.exp(neg_lam * dt_v[sl])
            for i in range(16):
                e = g * 16 + i
                wb = jnp.full((16,), w16[i], jnp.float32)
                for cc in range(D_FEAT // 16):
                    sl2 = pl.ds(cc * 16, 16)
                    rows_v[e, sl2] = rows_v[e, sl2] * wb
            return c2
        lax.fori_loop(0, CHUNK // 16, scale, 0)
        pltpu.sync_copy(rows_v, acc.at[dst_v], add=True)
        return carry
    lax.fori_loop(0, chunks_per_tile, chunk_body, 0)

    plsc.subcore_barrier()
    pltpu.sync_copy(acc.at[pl.ds(sid * rows_per_tile, rows_per_tile)],
                    out_hbm.at[cid, pl.ds(sid * rows_per_tile, rows_per_tile)])


def _sum2_body(a_ref, b_ref, o_ref):
    o_ref[...] = a_ref[...] + b_ref[...]


def kernel(x, edge_index, dt, norm, decay_lam):
    n_nodes, d_feat = x.shape
    n_edges = dt.shape[0]
    quant = NW * CHUNK
    n_pad = ((n_edges + quant - 1) // quant) * quant
    chunks_per_tile = n_pad // (NW * CHUNK)
    pad = n_pad - n_edges

    src = jnp.pad(edge_index[0].astype(jnp.int32), (0, pad))
    dst = jnp.pad(edge_index[1].astype(jnp.int32), (0, pad))
    dt_p = jnp.pad(dt, (0, pad))
    norm_p = jnp.pad(norm, (0, pad))
    lam_arr = jnp.full((16,), decay_lam, jnp.float32)

    n_acc = ((n_nodes + NS * CHUNK - 1) // (NS * CHUNK)) * (NS * CHUNK)
    rows_per_tile = n_acc // NS

    mesh = plsc.VectorSubcoreMesh(core_axis_name="c", subcore_axis_name="s")
    sc_kernel = pl.kernel(
        functools.partial(_sc_body, chunks_per_tile=chunks_per_tile,
                          rows_per_tile=rows_per_tile),
        out_type=jax.ShapeDtypeStruct((NC, n_acc, d_feat), jnp.float32),
        mesh=mesh,
        scratch_types=[
            pltpu.VMEM((CHUNK,), jnp.int32),
            pltpu.VMEM((CHUNK,), jnp.int32),
            pltpu.VMEM((CHUNK,), jnp.float32),
            pltpu.VMEM((CHUNK,), jnp.float32),
            pltpu.VMEM((CHUNK, D_FEAT), jnp.float32),
            pltpu.VMEM((16,), jnp.float32),
            pltpu.VMEM_SHARED((n_acc, d_feat), jnp.float32),
            pltpu.SemaphoreType.DMA,
        ],
    )
    partial = sc_kernel(x, src, dst, dt_p, norm_p, lam_arr)

    nb = 10
    out = pl.pallas_call(
        _sum2_body,
        out_shape=jax.ShapeDtypeStruct((n_nodes, d_feat), jnp.float32),
        grid=(nb,),
        in_specs=[pl.BlockSpec((n_nodes // nb, d_feat), lambda i: (i, 0))] * 2,
        out_specs=pl.BlockSpec((n_nodes // nb, d_feat), lambda i: (i, 0)),
    )(partial[0], partial[1])
    return out
